# baseline (device time: 74101 ns/iter reference)
import functools

import jax
import jax.numpy as jnp
from jax import lax
from jax.experimental import pallas as pl
from jax.experimental.pallas import tpu as pltpu

N_DEV = 32
N_STEPS = 5
N_LAYERS = 3


def kernel(x, Win0, Wout0, Win1, Wout1, Win2, Wout2):
    b, d = x.shape

    def body(x_ref, win0, wout0, win1, wout1, win2, wout2,
             out_ref, acc_ref, recv_ref, send_sems, recv_sems):
        my = lax.axis_index("i")
        partners = [my ^ (1 << k) for k in range(N_STEPS)]

        barrier_sem = pltpu.get_barrier_semaphore()
        for p in partners:
            pl.semaphore_signal(barrier_sem, inc=1, device_id=(p,),
                                device_id_type=pl.DeviceIdType.MESH)
        pl.semaphore_wait(barrier_sem, N_STEPS)

        wins = [win0, win1, win2]
        wouts = [wout0, wout1, wout2]
        xv = x_ref[...]
        for layer in range(N_LAYERS):
            h = jnp.dot(xv.astype(jnp.bfloat16),
                        wins[layer][...].astype(jnp.bfloat16),
                        preferred_element_type=jnp.float32)
            h = jnp.maximum(h, 0.0)
            part = jnp.dot(h.astype(jnp.bfloat16),
                           wouts[layer][...].astype(jnp.bfloat16),
                           preferred_element_type=jnp.float32)
            acc_ref[...] = part

            for k in range(N_STEPS):
                rdma = pltpu.make_async_remote_copy(
                    src_ref=acc_ref,
                    dst_ref=recv_ref.at[layer, k],
                    send_sem=send_sems.at[layer, k],
                    recv_sem=recv_sems.at[layer, k],
                    device_id=(partners[k],),
                    device_id_type=pl.DeviceIdType.MESH,
                )
                rdma.start()
                rdma.wait()
                acc_ref[...] = acc_ref[...] + recv_ref[layer, k]
            xv = acc_ref[...]

        out_ref[...] = xv

        @functools.partial(pl.run_scoped, sem=pltpu.SemaphoreType.REGULAR)
        def _(sem):
            for p in partners:
                pl.semaphore_signal(sem, inc=1, device_id=(p,),
                                    device_id_type=pl.DeviceIdType.MESH)
            pl.semaphore_wait(sem, N_STEPS)

    return pl.pallas_call(
        body,
        out_shape=jax.ShapeDtypeStruct((b, d), jnp.float32),
        in_specs=[pl.BlockSpec(memory_space=pltpu.VMEM)] * 7,
        out_specs=pl.BlockSpec(memory_space=pltpu.VMEM),
        scratch_shapes=[
            pltpu.VMEM((b, d), jnp.float32),
            pltpu.VMEM((N_LAYERS, N_STEPS, b, d), jnp.float32),
            pltpu.SemaphoreType.DMA((N_LAYERS, N_STEPS)),
            pltpu.SemaphoreType.DMA((N_LAYERS, N_STEPS)),
        ],
        compiler_params=pltpu.CompilerParams(collective_id=0),
    )(x, Win0, Wout0, Win1, Wout1, Win2, Wout2)
